# Initial kernel scaffold; baseline (speedup 1.0000x reference)
import jax
import jax.numpy as jnp
from jax.experimental import pallas as pl


def kernel(x, edge_index, W0, b0, W1, b1, W2, b2, Wl, bl):
    raise NotImplementedError("write your pallas kernel here")


# speedup vs baseline: 13.7082x; 13.7082x over previous
import functools

import jax
import jax.numpy as jnp
from jax import lax
from jax.experimental import pallas as pl
from jax.experimental.pallas import tpu as pltpu
from jax.experimental.pallas import tpu_sc as plsc

NC = 2
NS = 16
LANES = 16
CHUNK = 128


def _cdiv(a, b):
    return (a + b - 1) // b


def _deg_body(n_pad, e, dst_hbm, zeros_hbm, deg_hbm, dst_v, ones_v, deg_sh):
    c = lax.axis_index("c")
    s = lax.axis_index("s")
    rows = n_pad // NS

    for k in range(CHUNK // LANES):
        ones_v[pl.ds(k * LANES, LANES)] = jnp.ones((LANES,), jnp.float32)
    pltpu.sync_copy(zeros_hbm, deg_sh.at[pl.ds(s * rows, rows)])
    plsc.subcore_barrier()

    chunks_per_core = e // CHUNK // NC
    iters = _cdiv(chunks_per_core, NS)

    def body(j, carry):
        ch = s + NS * j

        @pl.when(ch < chunks_per_core)
        def _():
            base = (c * chunks_per_core + ch) * CHUNK
            pltpu.sync_copy(dst_hbm.at[pl.ds(base, CHUNK)], dst_v)
            pltpu.sync_copy(ones_v, deg_sh.at[dst_v], add=True)

        return carry

    lax.fori_loop(0, iters, body, 0)
    plsc.subcore_barrier()
    pltpu.sync_copy(deg_sh.at[pl.ds(s * rows, rows)],
                    deg_hbm.at[c, pl.ds(s * rows, rows)])


def _make_deg_kernel(n_pad, e):
    mesh = plsc.VectorSubcoreMesh(core_axis_name="c", subcore_axis_name="s")
    return pl.kernel(
        functools.partial(_deg_body, n_pad, e),
        out_type=jax.ShapeDtypeStruct((NC, n_pad), jnp.float32),
        mesh=mesh,
        scratch_types=[
            pltpu.VMEM((CHUNK,), jnp.int32),
            pltpu.VMEM((CHUNK,), jnp.float32),
            pltpu.VMEM_SHARED((n_pad,), jnp.float32),
        ],
    )


def _agg_body(n_pad, d, e, g_hbm, src_hbm, dst_hbm, zeros_hbm, out_hbm,
              src_v, dst_v, rows_v, acc_sh, sem):
    c = lax.axis_index("c")
    s = lax.axis_index("s")
    rows = n_pad // NS

    pltpu.sync_copy(zeros_hbm, acc_sh.at[pl.ds(s * rows, rows)])
    plsc.subcore_barrier()

    total_chunks = e // CHUNK
    nw = NC * NS
    w = s * NC + c
    iters = _cdiv(total_chunks, nw)

    def body(j, carry):
        ch = w + nw * j

        @pl.when(ch < total_chunks)
        def _():
            base = ch * CHUNK
            pltpu.sync_copy(src_hbm.at[pl.ds(base, CHUNK)], src_v)
            pltpu.sync_copy(dst_hbm.at[pl.ds(base, CHUNK)], dst_v)
            pltpu.async_copy(g_hbm.at[src_v], rows_v, sem).wait()
            pltpu.sync_copy(rows_v, acc_sh.at[dst_v], add=True)

        return carry

    lax.fori_loop(0, iters, body, 0)
    plsc.subcore_barrier()
    pltpu.sync_copy(acc_sh.at[pl.ds(s * rows, rows)],
                    out_hbm.at[c, pl.ds(s * rows, rows)])


def _make_agg_kernel(n, n_pad, d, e):
    mesh = plsc.VectorSubcoreMesh(core_axis_name="c", subcore_axis_name="s")
    return pl.kernel(
        functools.partial(_agg_body, n_pad, d, e),
        out_type=jax.ShapeDtypeStruct((NC, n_pad, d), jnp.float32),
        mesh=mesh,
        scratch_types=[
            pltpu.VMEM((CHUNK,), jnp.int32),
            pltpu.VMEM((CHUNK,), jnp.int32),
            pltpu.VMEM((CHUNK, d), jnp.float32),
            pltpu.VMEM_SHARED((n_pad, d), jnp.float32),
            pltpu.SemaphoreType.DMA,
        ],
    )


def _pre_body(x_ref, w_ref, degc_ref, t_ref, g_ref):
    dis = lax.rsqrt(1.0 + degc_ref[...])
    t = jnp.dot(x_ref[...], w_ref[...], preferred_element_type=jnp.float32)
    t_ref[...] = t
    g_ref[...] = dis * t


def _mid_body(agg_ref, t_ref, degc_ref, b_ref, w_ref, tn_ref, gn_ref, h_ref):
    dis = lax.rsqrt(1.0 + degc_ref[...])
    a = agg_ref[0] + agg_ref[1]
    h = jnp.maximum(dis * a + dis * dis * t_ref[...] + b_ref[...], 0.0)
    tn = jnp.dot(h, w_ref[...], preferred_element_type=jnp.float32)
    tn_ref[...] = tn
    gn_ref[...] = dis * tn
    h_ref[...] = h


def _post_body(agg_ref, t_ref, degc_ref, b_ref, h1_ref, h2_ref, wl_ref, bl_ref,
               o_ref):
    dis = lax.rsqrt(1.0 + degc_ref[...])
    a = agg_ref[0] + agg_ref[1]
    h3 = jnp.maximum(dis * a + dis * dis * t_ref[...] + b_ref[...], 0.0)
    hcat = jnp.concatenate([h1_ref[...], h2_ref[...], h3], axis=1)
    o = jnp.dot(hcat, wl_ref[...], preferred_element_type=jnp.float32)
    o = o + bl_ref[...]
    m = jnp.max(o, axis=1, keepdims=True)
    lse = jnp.log(jnp.sum(jnp.exp(o - m), axis=1, keepdims=True)) + m
    o_ref[...] = o - lse


def kernel(x, edge_index, W0, b0, W1, b1, W2, b2, Wl, bl):
    n, d_in = x.shape
    e = edge_index.shape[1]
    d = W0.shape[1]
    d_out = Wl.shape[1]
    n_pad = _cdiv(n, NC * NS * LANES) * (NC * NS * LANES)
    bn = 2000
    grid = n // bn

    src = edge_index[0]
    dst = edge_index[1]
    zeros_row = jnp.zeros((n_pad // NS,), jnp.float32)
    zeros_blk = jnp.zeros((n_pad // NS, d), jnp.float32)

    deg2 = _make_deg_kernel(n_pad, e)(dst, zeros_row)
    degc = (deg2[0, :n] + deg2[1, :n]).reshape(n, 1)

    agg_fn = _make_agg_kernel(n, n_pad, d, e)

    row_blk = lambda i: (i, 0)
    full_w = lambda a, b_: pl.BlockSpec((a, b_), lambda i: (0, 0))
    vec = lambda L: pl.BlockSpec((L,), lambda i: (0,))
    nd_blk = pl.BlockSpec((bn, d), row_blk)
    col_blk = pl.BlockSpec((bn, 1), row_blk)
    agg_blk = pl.BlockSpec((NC, bn, d), lambda i: (0, i, 0))

    t0, g0 = pl.pallas_call(
        _pre_body,
        grid=(grid,),
        in_specs=[nd_blk, full_w(d_in, d), col_blk],
        out_specs=[nd_blk, nd_blk],
        out_shape=[jax.ShapeDtypeStruct((n, d), jnp.float32)] * 2,
    )(x, W0, degc)

    mid = pl.pallas_call(
        _mid_body,
        grid=(grid,),
        in_specs=[agg_blk, nd_blk, col_blk, vec(d), full_w(d, d)],
        out_specs=[nd_blk, nd_blk, nd_blk],
        out_shape=[jax.ShapeDtypeStruct((n, d), jnp.float32)] * 3,
    )

    agg1 = agg_fn(g0, src, dst, zeros_blk)
    t1, g1, h1 = mid(agg1, t0, degc, b0, W1)

    agg2 = agg_fn(g1, src, dst, zeros_blk)
    t2, g2, h2 = mid(agg2, t1, degc, b1, W2)

    agg3 = agg_fn(g2, src, dst, zeros_blk)

    out = pl.pallas_call(
        _post_body,
        grid=(grid,),
        in_specs=[agg_blk, nd_blk, col_blk, vec(d), nd_blk, nd_blk,
                  full_w(3 * d, d_out), vec(d_out)],
        out_specs=pl.BlockSpec((bn, d_out), row_blk),
        out_shape=jax.ShapeDtypeStruct((n, d_out), jnp.float32),
    )(agg3, t2, degc, b2, h1, h2, Wl, bl)

    return out


# speedup vs baseline: 25.6401x; 1.8704x over previous
import functools

import jax
import jax.numpy as jnp
from jax import lax
from jax.experimental import pallas as pl
from jax.experimental.pallas import tpu as pltpu
from jax.experimental.pallas import tpu_sc as plsc

NC = 2
NS = 16
LANES = 16
CHUNK = 128


def _cdiv(a, b):
    return (a + b - 1) // b


def _deg_body(n_pad, e, dst_hbm, zeros_hbm, deg_hbm, dst_v, ones_v, deg_sh):
    c = lax.axis_index("c")
    s = lax.axis_index("s")
    rows = n_pad // NS

    for k in range(CHUNK // LANES):
        ones_v[pl.ds(k * LANES, LANES)] = jnp.ones((LANES,), jnp.float32)
    pltpu.sync_copy(zeros_hbm, deg_sh.at[pl.ds(s * rows, rows)])
    plsc.subcore_barrier()

    chunks_per_core = e // CHUNK // NC
    iters = _cdiv(chunks_per_core, NS)

    def body(j, carry):
        ch = s + NS * j

        @pl.when(ch < chunks_per_core)
        def _():
            base = (c * chunks_per_core + ch) * CHUNK
            pltpu.sync_copy(dst_hbm.at[pl.ds(base, CHUNK)], dst_v)
            pltpu.sync_copy(ones_v, deg_sh.at[dst_v], add=True)

        return carry

    lax.fori_loop(0, iters, body, 0)
    plsc.subcore_barrier()
    pltpu.sync_copy(deg_sh.at[pl.ds(s * rows, rows)],
                    deg_hbm.at[c, 0, pl.ds(s * rows, rows)])


def _make_deg_kernel(n_pad, e):
    mesh = plsc.VectorSubcoreMesh(core_axis_name="c", subcore_axis_name="s")
    return pl.kernel(
        functools.partial(_deg_body, n_pad, e),
        out_type=jax.ShapeDtypeStruct((NC, 1, n_pad), jnp.float32),
        mesh=mesh,
        scratch_types=[
            pltpu.VMEM((CHUNK,), jnp.int32),
            pltpu.VMEM((CHUNK,), jnp.float32),
            pltpu.VMEM_SHARED((n_pad,), jnp.float32),
        ],
    )


def _agg_body(n_pad, d, e, g_hbm, src2_hbm, src1_hbm, dst1_hbm, zeros_hbm,
              out_hbm, src_v, dst_a, dst_b, rows0, rows1, acc_sh, sem0, sem1):
    c = lax.axis_index("c")
    s = lax.axis_index("s")
    rows = n_pad // NS

    total_chunks = e // CHUNK
    nw = NC * NS
    w = s * NC + c
    nbase = total_chunks // nw
    rem = total_chunks - nbase * nw

    pltpu.sync_copy(zeros_hbm, acc_sh.at[pl.ds(s * rows, rows)])
    pltpu.sync_copy(src2_hbm.at[pl.ds(w * nbase, nbase)],
                    src_v.at[pl.ds(0, nbase)])

    @pl.when(w < rem)
    def _():
        pltpu.sync_copy(src1_hbm.at[pl.ds((nbase * nw + w) * CHUNK, CHUNK)],
                        src_v.at[nbase, 0])

    plsc.subcore_barrier()

    rbufs = (rows0, rows1)
    dbufs = (dst_a, dst_b)
    sems = (sem0, sem1)

    def chunk_idx(j):
        return jnp.where(j < nbase, w * nbase + j, nbase * nw + w)

    def fire(j, b):
        pltpu.async_copy(g_hbm.at[src_v.at[j, 0]], rbufs[b], sems[b])
        pltpu.async_copy(dst1_hbm.at[pl.ds(chunk_idx(j) * CHUNK, CHUNK)],
                         dbufs[b], sems[b])

    def drain_and_scatter(j, b):
        pltpu.make_async_copy(g_hbm.at[src_v.at[j, 0]], rbufs[b],
                              sems[b]).wait()
        pltpu.make_async_copy(dst1_hbm.at[pl.ds(0, CHUNK)], dbufs[b],
                              sems[b]).wait()
        pltpu.sync_copy(rbufs[b], acc_sh.at[dbufs[b]], add=True)

    fire(0, 0)

    def pair(jj, carry):
        for b in (0, 1):
            j = 2 * jj + b
            nxt = j + 1

            @pl.when((nxt < nbase) | ((nxt == nbase) & (w < rem)))
            def _():
                fire(nxt, 1 - b)

            drain_and_scatter(j, b)
        return carry

    assert nbase % 2 == 0
    lax.fori_loop(0, nbase // 2, pair, 0)

    @pl.when(w < rem)
    def _():
        drain_and_scatter(nbase, 0)

    plsc.subcore_barrier()
    pltpu.sync_copy(acc_sh.at[pl.ds(s * rows, rows)],
                    out_hbm.at[c, pl.ds(s * rows, rows)])


def _make_agg_kernel(n, n_pad, d, e):
    mesh = plsc.VectorSubcoreMesh(core_axis_name="c", subcore_axis_name="s")
    nchunk_t = _cdiv(e // CHUNK, NC * NS)
    return pl.kernel(
        functools.partial(_agg_body, n_pad, d, e),
        out_type=jax.ShapeDtypeStruct((NC, n_pad, d), jnp.float32),
        mesh=mesh,
        scratch_types=[
            pltpu.VMEM((nchunk_t, 1, CHUNK), jnp.int32),
            pltpu.VMEM((CHUNK,), jnp.int32),
            pltpu.VMEM((CHUNK,), jnp.int32),
            pltpu.VMEM((CHUNK, d), jnp.float32),
            pltpu.VMEM((CHUNK, d), jnp.float32),
            pltpu.VMEM_SHARED((n_pad, d), jnp.float32),
            pltpu.SemaphoreType.DMA,
            pltpu.SemaphoreType.DMA,
        ],
    )


def _pre_body(x_ref, w_ref, degc_ref, t_ref, g_ref):
    dis = lax.rsqrt(1.0 + degc_ref[...])
    t = jnp.dot(x_ref[...], w_ref[...], preferred_element_type=jnp.float32)
    t_ref[...] = t
    g_ref[...] = dis * t


def _mid_body(agg_ref, t_ref, degc_ref, b_ref, w_ref, tn_ref, gn_ref, h_ref):
    dis = lax.rsqrt(1.0 + degc_ref[...])
    a = agg_ref[0] + agg_ref[1]
    h = jnp.maximum(dis * a + dis * dis * t_ref[...] + b_ref[...], 0.0)
    tn = jnp.dot(h, w_ref[...], preferred_element_type=jnp.float32)
    tn_ref[...] = tn
    gn_ref[...] = dis * tn
    h_ref[...] = h


def _post_body(agg_ref, t_ref, degc_ref, b_ref, h1_ref, h2_ref, wl_ref, bl_ref,
               o_ref):
    dis = lax.rsqrt(1.0 + degc_ref[...])
    a = agg_ref[0] + agg_ref[1]
    h3 = jnp.maximum(dis * a + dis * dis * t_ref[...] + b_ref[...], 0.0)
    hcat = jnp.concatenate([h1_ref[...], h2_ref[...], h3], axis=1)
    o = jnp.dot(hcat, wl_ref[...], preferred_element_type=jnp.float32)
    o = o + bl_ref[...]
    m = jnp.max(o, axis=1, keepdims=True)
    lse = jnp.log(jnp.sum(jnp.exp(o - m), axis=1, keepdims=True)) + m
    o_ref[...] = o - lse


def kernel(x, edge_index, W0, b0, W1, b1, W2, b2, Wl, bl):
    n, d_in = x.shape
    e = edge_index.shape[1]
    d = W0.shape[1]
    d_out = Wl.shape[1]
    n_pad = _cdiv(n, 128 * NS) * (128 * NS)
    bn = 2000
    grid = n // bn

    src = edge_index[0]
    dst = edge_index[1]
    src2 = src.reshape(e // CHUNK, 1, CHUNK)
    zeros_row = jnp.zeros((n_pad // NS,), jnp.float32)
    zeros_blk = jnp.zeros((n_pad // NS, d), jnp.float32)

    deg2 = _make_deg_kernel(n_pad, e)(dst, zeros_row)
    degc = (deg2[0, 0, :n] + deg2[1, 0, :n]).reshape(n, 1)

    agg_fn = _make_agg_kernel(n, n_pad, d, e)

    row_blk = lambda i: (i, 0)
    full_w = lambda a, b_: pl.BlockSpec((a, b_), lambda i: (0, 0))
    vec = lambda L: pl.BlockSpec((L,), lambda i: (0,))
    nd_blk = pl.BlockSpec((bn, d), row_blk)
    col_blk = pl.BlockSpec((bn, 1), row_blk)
    agg_blk = pl.BlockSpec((NC, bn, d), lambda i: (0, i, 0))

    t0, g0 = pl.pallas_call(
        _pre_body,
        grid=(grid,),
        in_specs=[nd_blk, full_w(d_in, d), col_blk],
        out_specs=[nd_blk, nd_blk],
        out_shape=[jax.ShapeDtypeStruct((n, d), jnp.float32)] * 2,
    )(x, W0, degc)

    mid = pl.pallas_call(
        _mid_body,
        grid=(grid,),
        in_specs=[agg_blk, nd_blk, col_blk, vec(d), full_w(d, d)],
        out_specs=[nd_blk, nd_blk, nd_blk],
        out_shape=[jax.ShapeDtypeStruct((n, d), jnp.float32)] * 3,
    )

    agg1 = agg_fn(g0, src2, src, dst, zeros_blk)
    t1, g1, h1 = mid(agg1, t0, degc, b0, W1)

    agg2 = agg_fn(g1, src2, src, dst, zeros_blk)
    t2, g2, h2 = mid(agg2, t1, degc, b1, W2)

    agg3 = agg_fn(g2, src2, src, dst, zeros_blk)

    out = pl.pallas_call(
        _post_body,
        grid=(grid,),
        in_specs=[agg_blk, nd_blk, col_blk, vec(d), nd_blk, nd_blk,
                  full_w(3 * d, d_out), vec(d_out)],
        out_specs=pl.BlockSpec((bn, d_out), row_blk),
        out_shape=jax.ShapeDtypeStruct((n, d_out), jnp.float32),
    )(agg3, t2, degc, b2, h1, h2, Wl, bl)

    return out


# speedup vs baseline: 28.1746x; 1.0988x over previous
import functools

import jax
import jax.numpy as jnp
from jax import lax
from jax.experimental import pallas as pl
from jax.experimental.pallas import tpu as pltpu
from jax.experimental.pallas import tpu_sc as plsc

NC = 2
NS = 16
LANES = 16
CHUNK = 128


def _cdiv(a, b):
    return (a + b - 1) // b


def _deg_body(n_pad, e, dst2_hbm, dst1_hbm, zeros_hbm, deg_hbm,
              idx_v, ones_v, deg_sh, sem):
    c = lax.axis_index("c")
    s = lax.axis_index("s")
    rows = n_pad // NS

    total_chunks = e // CHUNK
    nw = NC * NS
    w = s * NC + c
    nbase = total_chunks // nw
    rem = total_chunks - nbase * nw

    for k in range(CHUNK // LANES):
        ones_v[pl.ds(k * LANES, LANES)] = jnp.ones((LANES,), jnp.float32)
    pltpu.sync_copy(zeros_hbm, deg_sh.at[pl.ds(s * rows, rows)])
    pltpu.sync_copy(dst2_hbm.at[pl.ds(w * nbase, nbase)],
                    idx_v.at[pl.ds(0, nbase)])

    @pl.when(w < rem)
    def _():
        pltpu.sync_copy(dst1_hbm.at[pl.ds((nbase * nw + w) * CHUNK, CHUNK)],
                        idx_v.at[nbase, 0])

    plsc.subcore_barrier()

    ntot = nbase + 1
    def fire(j, carry):
        @pl.when((j < nbase) | (w < rem))
        def _():
            pltpu.async_copy(ones_v, deg_sh.at[idx_v.at[j, 0]], sem,
                             add=True)
        return carry

    lax.fori_loop(0, ntot, fire, 0)

    def drain(j, carry):
        @pl.when((j < nbase) | (w < rem))
        def _():
            pltpu.make_async_copy(ones_v, deg_sh.at[idx_v.at[j, 0]],
                                  sem).wait()
        return carry

    lax.fori_loop(0, ntot, drain, 0)
    plsc.subcore_barrier()
    pltpu.sync_copy(deg_sh.at[pl.ds(s * rows, rows)],
                    deg_hbm.at[c, 0, pl.ds(s * rows, rows)])


def _make_deg_kernel(n_pad, e):
    mesh = plsc.VectorSubcoreMesh(core_axis_name="c", subcore_axis_name="s")
    nchunk_t = _cdiv(e // CHUNK, NC * NS)
    return pl.kernel(
        functools.partial(_deg_body, n_pad, e),
        out_type=jax.ShapeDtypeStruct((NC, 1, n_pad), jnp.float32),
        mesh=mesh,
        scratch_types=[
            pltpu.VMEM((nchunk_t, 1, CHUNK), jnp.int32),
            pltpu.VMEM((CHUNK,), jnp.float32),
            pltpu.VMEM_SHARED((n_pad,), jnp.float32),
            pltpu.SemaphoreType.DMA,
        ],
    )


def _agg_body(n_pad, d, e, g_hbm, src2_hbm, src1_hbm, dst1_hbm, zeros_hbm,
              out_hbm, src_v, dst_a, dst_b, rows0, rows1, acc_sh, sem0, sem1):
    c = lax.axis_index("c")
    s = lax.axis_index("s")
    rows = n_pad // NS

    total_chunks = e // CHUNK
    nw = NC * NS
    w = s * NC + c
    nbase = total_chunks // nw
    rem = total_chunks - nbase * nw

    pltpu.sync_copy(zeros_hbm, acc_sh.at[pl.ds(s * rows, rows)])
    pltpu.sync_copy(src2_hbm.at[pl.ds(w * nbase, nbase)],
                    src_v.at[pl.ds(0, nbase)])

    @pl.when(w < rem)
    def _():
        pltpu.sync_copy(src1_hbm.at[pl.ds((nbase * nw + w) * CHUNK, CHUNK)],
                        src_v.at[nbase, 0])

    plsc.subcore_barrier()

    rbufs = (rows0, rows1)
    dbufs = (dst_a, dst_b)
    sems = (sem0, sem1)

    def chunk_idx(j):
        return jnp.where(j < nbase, w * nbase + j, nbase * nw + w)

    def fire(j, b):
        pltpu.async_copy(g_hbm.at[src_v.at[j, 0]], rbufs[b], sems[b])
        pltpu.async_copy(dst1_hbm.at[pl.ds(chunk_idx(j) * CHUNK, CHUNK)],
                         dbufs[b], sems[b])

    def drain_and_scatter(j, b):
        pltpu.make_async_copy(g_hbm.at[src_v.at[j, 0]], rbufs[b],
                              sems[b]).wait()
        pltpu.make_async_copy(dst1_hbm.at[pl.ds(0, CHUNK)], dbufs[b],
                              sems[b]).wait()
        pltpu.sync_copy(rbufs[b], acc_sh.at[dbufs[b]], add=True)

    fire(0, 0)

    def pair(jj, carry):
        for b in (0, 1):
            j = 2 * jj + b
            nxt = j + 1

            @pl.when((nxt < nbase) | ((nxt == nbase) & (w < rem)))
            def _():
                fire(nxt, 1 - b)

            drain_and_scatter(j, b)
        return carry

    assert nbase % 2 == 0
    lax.fori_loop(0, nbase // 2, pair, 0)

    @pl.when(w < rem)
    def _():
        drain_and_scatter(nbase, 0)

    plsc.subcore_barrier()
    pltpu.sync_copy(acc_sh.at[pl.ds(s * rows, rows)],
                    out_hbm.at[c, pl.ds(s * rows, rows)])


def _make_agg_kernel(n, n_pad, d, e):
    mesh = plsc.VectorSubcoreMesh(core_axis_name="c", subcore_axis_name="s")
    nchunk_t = _cdiv(e // CHUNK, NC * NS)
    return pl.kernel(
        functools.partial(_agg_body, n_pad, d, e),
        out_type=jax.ShapeDtypeStruct((NC, n_pad, d), jnp.float32),
        mesh=mesh,
        scratch_types=[
            pltpu.VMEM((nchunk_t, 1, CHUNK), jnp.int32),
            pltpu.VMEM((CHUNK,), jnp.int32),
            pltpu.VMEM((CHUNK,), jnp.int32),
            pltpu.VMEM((CHUNK, d), jnp.float32),
            pltpu.VMEM((CHUNK, d), jnp.float32),
            pltpu.VMEM_SHARED((n_pad, d), jnp.float32),
            pltpu.SemaphoreType.DMA,
            pltpu.SemaphoreType.DMA,
        ],
    )


def _pre_body(x_ref, w_ref, degc_ref, t_ref, g_ref):
    dis = lax.rsqrt(1.0 + degc_ref[...])
    t = jnp.dot(x_ref[...], w_ref[...], preferred_element_type=jnp.float32)
    t_ref[...] = t
    g_ref[...] = dis * t


def _mid_body(agg_ref, t_ref, degc_ref, b_ref, w_ref, tn_ref, gn_ref, h_ref):
    dis = lax.rsqrt(1.0 + degc_ref[...])
    a = agg_ref[0] + agg_ref[1]
    h = jnp.maximum(dis * a + dis * dis * t_ref[...] + b_ref[...], 0.0)
    tn = jnp.dot(h, w_ref[...], preferred_element_type=jnp.float32)
    tn_ref[...] = tn
    gn_ref[...] = dis * tn
    h_ref[...] = h


def _post_body(agg_ref, t_ref, degc_ref, b_ref, h1_ref, h2_ref, wl_ref, bl_ref,
               o_ref):
    dis = lax.rsqrt(1.0 + degc_ref[...])
    a = agg_ref[0] + agg_ref[1]
    h3 = jnp.maximum(dis * a + dis * dis * t_ref[...] + b_ref[...], 0.0)
    hcat = jnp.concatenate([h1_ref[...], h2_ref[...], h3], axis=1)
    o = jnp.dot(hcat, wl_ref[...], preferred_element_type=jnp.float32)
    o = o + bl_ref[...]
    m = jnp.max(o, axis=1, keepdims=True)
    lse = jnp.log(jnp.sum(jnp.exp(o - m), axis=1, keepdims=True)) + m
    o_ref[...] = o - lse


def kernel(x, edge_index, W0, b0, W1, b1, W2, b2, Wl, bl):
    n, d_in = x.shape
    e = edge_index.shape[1]
    d = W0.shape[1]
    d_out = Wl.shape[1]
    n_pad = _cdiv(n, 128 * NS) * (128 * NS)
    bn = 2000
    grid = n // bn

    src = edge_index[0]
    dst = edge_index[1]
    src2 = src.reshape(e // CHUNK, 1, CHUNK)
    dst2 = dst.reshape(e // CHUNK, 1, CHUNK)
    zeros_row = jnp.zeros((n_pad // NS,), jnp.float32)
    zeros_blk = jnp.zeros((n_pad // NS, d), jnp.float32)

    deg2 = _make_deg_kernel(n_pad, e)(dst2, dst, zeros_row)
    degc = (deg2[0, 0, :n] + deg2[1, 0, :n]).reshape(n, 1)

    agg_fn = _make_agg_kernel(n, n_pad, d, e)

    row_blk = lambda i: (i, 0)
    full_w = lambda a, b_: pl.BlockSpec((a, b_), lambda i: (0, 0))
    vec = lambda L: pl.BlockSpec((L,), lambda i: (0,))
    nd_blk = pl.BlockSpec((bn, d), row_blk)
    col_blk = pl.BlockSpec((bn, 1), row_blk)
    agg_blk = pl.BlockSpec((NC, bn, d), lambda i: (0, i, 0))

    t0, g0 = pl.pallas_call(
        _pre_body,
        grid=(grid,),
        in_specs=[nd_blk, full_w(d_in, d), col_blk],
        out_specs=[nd_blk, nd_blk],
        out_shape=[jax.ShapeDtypeStruct((n, d), jnp.float32)] * 2,
    )(x, W0, degc)

    mid = pl.pallas_call(
        _mid_body,
        grid=(grid,),
        in_specs=[agg_blk, nd_blk, col_blk, vec(d), full_w(d, d)],
        out_specs=[nd_blk, nd_blk, nd_blk],
        out_shape=[jax.ShapeDtypeStruct((n, d), jnp.float32)] * 3,
    )

    agg1 = agg_fn(g0, src2, src, dst, zeros_blk)
    t1, g1, h1 = mid(agg1, t0, degc, b0, W1)

    agg2 = agg_fn(g1, src2, src, dst, zeros_blk)
    t2, g2, h2 = mid(agg2, t1, degc, b1, W2)

    agg3 = agg_fn(g2, src2, src, dst, zeros_blk)

    out = pl.pallas_call(
        _post_body,
        grid=(grid,),
        in_specs=[agg_blk, nd_blk, col_blk, vec(d), nd_blk, nd_blk,
                  full_w(3 * d, d_out), vec(d_out)],
        out_specs=pl.BlockSpec((bn, d_out), row_blk),
        out_shape=jax.ShapeDtypeStruct((n, d_out), jnp.float32),
    )(agg3, t2, degc, b2, h1, h2, Wl, bl)

    return out


# speedup vs baseline: 28.8319x; 1.0233x over previous
import functools

import jax
import jax.numpy as jnp
from jax import lax
from jax.experimental import pallas as pl
from jax.experimental.pallas import tpu as pltpu
from jax.experimental.pallas import tpu_sc as plsc

NC = 2
NS = 16
LANES = 16
CHUNK = 128


def _cdiv(a, b):
    return (a + b - 1) // b


def _deg_body(n_pad, e, dst2_hbm, dst1_hbm, zeros_hbm, deg_hbm,
              idx_v, ones_v, deg_sh, sem):
    c = lax.axis_index("c")
    s = lax.axis_index("s")
    rows = n_pad // NS

    total_chunks = e // CHUNK
    nw = NC * NS
    w = s * NC + c
    nbase = total_chunks // nw
    rem = total_chunks - nbase * nw

    for k in range(CHUNK // LANES):
        ones_v[pl.ds(k * LANES, LANES)] = jnp.ones((LANES,), jnp.float32)
    pltpu.sync_copy(zeros_hbm, deg_sh.at[pl.ds(s * rows, rows)])
    pltpu.sync_copy(dst2_hbm.at[pl.ds(w * nbase, nbase)],
                    idx_v.at[pl.ds(0, nbase)])

    @pl.when(w < rem)
    def _():
        pltpu.sync_copy(dst1_hbm.at[pl.ds((nbase * nw + w) * CHUNK, CHUNK)],
                        idx_v.at[nbase, 0])

    plsc.subcore_barrier()

    ntot = nbase + 1
    def fire(j, carry):
        @pl.when((j < nbase) | (w < rem))
        def _():
            pltpu.async_copy(ones_v, deg_sh.at[idx_v.at[j, 0]], sem,
                             add=True)
        return carry

    lax.fori_loop(0, ntot, fire, 0)

    def drain(j, carry):
        @pl.when((j < nbase) | (w < rem))
        def _():
            pltpu.make_async_copy(ones_v, deg_sh.at[idx_v.at[j, 0]],
                                  sem).wait()
        return carry

    lax.fori_loop(0, ntot, drain, 0)
    plsc.subcore_barrier()
    pltpu.sync_copy(deg_sh.at[pl.ds(s * rows, rows)],
                    deg_hbm.at[c, 0, pl.ds(s * rows, rows)])


def _make_deg_kernel(n_pad, e):
    mesh = plsc.VectorSubcoreMesh(core_axis_name="c", subcore_axis_name="s")
    nchunk_t = _cdiv(e // CHUNK, NC * NS)
    return pl.kernel(
        functools.partial(_deg_body, n_pad, e),
        out_type=jax.ShapeDtypeStruct((NC, 1, n_pad), jnp.float32),
        mesh=mesh,
        scratch_types=[
            pltpu.VMEM((nchunk_t, 1, CHUNK), jnp.int32),
            pltpu.VMEM((CHUNK,), jnp.float32),
            pltpu.VMEM_SHARED((n_pad,), jnp.float32),
            pltpu.SemaphoreType.DMA,
        ],
    )


def _agg_body(n_pad, d, e, g_hbm, src2_hbm, src1_hbm, dst1_hbm, zeros_hbm,
              out_hbm, src_v, dst_a, dst_b, rows0, rows1, acc_sh, sem0, sem1):
    c = lax.axis_index("c")
    s = lax.axis_index("s")
    rows = n_pad // NS

    total_chunks = e // CHUNK
    nw = NC * NS
    w = s * NC + c
    nbase = total_chunks // nw
    rem = total_chunks - nbase * nw

    pltpu.async_copy(zeros_hbm, acc_sh.at[pl.ds(s * rows, rows)], sem0)
    pltpu.async_copy(src2_hbm.at[pl.ds(w * nbase, nbase)],
                     src_v.at[pl.ds(0, nbase)], sem1)

    @pl.when(w < rem)
    def _():
        pltpu.async_copy(src1_hbm.at[pl.ds((nbase * nw + w) * CHUNK, CHUNK)],
                         src_v.at[nbase, 0], sem1)

    pltpu.make_async_copy(zeros_hbm, acc_sh.at[pl.ds(s * rows, rows)],
                          sem0).wait()
    pltpu.make_async_copy(src2_hbm.at[pl.ds(w * nbase, nbase)],
                          src_v.at[pl.ds(0, nbase)], sem1).wait()

    @pl.when(w < rem)
    def _():
        pltpu.make_async_copy(
            src1_hbm.at[pl.ds((nbase * nw + w) * CHUNK, CHUNK)],
            src_v.at[nbase, 0], sem1).wait()

    plsc.subcore_barrier()

    rbufs = (rows0, rows1)
    dbufs = (dst_a, dst_b)
    sems = (sem0, sem1)

    def chunk_idx(j):
        return jnp.where(j < nbase, w * nbase + j, nbase * nw + w)

    def fire(j, b):
        pltpu.async_copy(g_hbm.at[src_v.at[j, 0]], rbufs[b], sems[b])
        pltpu.async_copy(dst1_hbm.at[pl.ds(chunk_idx(j) * CHUNK, CHUNK)],
                         dbufs[b], sems[b])

    def drain_and_scatter(j, b):
        pltpu.make_async_copy(g_hbm.at[src_v.at[j, 0]], rbufs[b],
                              sems[b]).wait()
        pltpu.make_async_copy(dst1_hbm.at[pl.ds(0, CHUNK)], dbufs[b],
                              sems[b]).wait()
        pltpu.sync_copy(rbufs[b], acc_sh.at[dbufs[b]], add=True)

    fire(0, 0)

    def pair(jj, carry):
        for b in (0, 1):
            j = 2 * jj + b
            nxt = j + 1

            @pl.when((nxt < nbase) | ((nxt == nbase) & (w < rem)))
            def _():
                fire(nxt, 1 - b)

            drain_and_scatter(j, b)
        return carry

    assert nbase % 2 == 0
    lax.fori_loop(0, nbase // 2, pair, 0)

    @pl.when(w < rem)
    def _():
        drain_and_scatter(nbase, 0)

    plsc.subcore_barrier()
    pltpu.sync_copy(acc_sh.at[pl.ds(s * rows, rows)],
                    out_hbm.at[c, pl.ds(s * rows, rows)])


def _make_agg_kernel(n, n_pad, d, e):
    mesh = plsc.VectorSubcoreMesh(core_axis_name="c", subcore_axis_name="s")
    nchunk_t = _cdiv(e // CHUNK, NC * NS)
    return pl.kernel(
        functools.partial(_agg_body, n_pad, d, e),
        out_type=jax.ShapeDtypeStruct((NC, n_pad, d), jnp.float32),
        mesh=mesh,
        scratch_types=[
            pltpu.VMEM((nchunk_t, 1, CHUNK), jnp.int32),
            pltpu.VMEM((CHUNK,), jnp.int32),
            pltpu.VMEM((CHUNK,), jnp.int32),
            pltpu.VMEM((CHUNK, d), jnp.float32),
            pltpu.VMEM((CHUNK, d), jnp.float32),
            pltpu.VMEM_SHARED((n_pad, d), jnp.float32),
            pltpu.SemaphoreType.DMA,
            pltpu.SemaphoreType.DMA,
        ],
    )


def _pre_body(x_ref, w_ref, degc_ref, g_ref):
    dis = lax.rsqrt(1.0 + degc_ref[...])
    t = jnp.dot(x_ref[...], w_ref[...], preferred_element_type=jnp.float32)
    g_ref[...] = dis * t


def _mid1_body(d, agg_ref, g_ref, degc_ref, b_ref, w_ref, wl_ref,
               gn_ref, o_ref):
    dis = lax.rsqrt(1.0 + degc_ref[...])
    a = agg_ref[0] + agg_ref[1]
    h = jnp.maximum(dis * (a + g_ref[...]) + b_ref[...], 0.0)
    tn = jnp.dot(h, w_ref[...], preferred_element_type=jnp.float32)
    gn_ref[...] = dis * tn
    o_ref[...] = jnp.dot(h, wl_ref[0:d, :],
                         preferred_element_type=jnp.float32)


def _mid2_body(d, agg_ref, g_ref, degc_ref, b_ref, w_ref, wl_ref, oin_ref,
               gn_ref, o_ref):
    dis = lax.rsqrt(1.0 + degc_ref[...])
    a = agg_ref[0] + agg_ref[1]
    h = jnp.maximum(dis * (a + g_ref[...]) + b_ref[...], 0.0)
    tn = jnp.dot(h, w_ref[...], preferred_element_type=jnp.float32)
    gn_ref[...] = dis * tn
    o_ref[...] = oin_ref[...] + jnp.dot(h, wl_ref[d:2 * d, :],
                                        preferred_element_type=jnp.float32)


def _post_body(d, agg_ref, g_ref, degc_ref, b_ref, wl_ref, bl_ref, oin_ref,
               o_ref):
    dis = lax.rsqrt(1.0 + degc_ref[...])
    a = agg_ref[0] + agg_ref[1]
    h3 = jnp.maximum(dis * (a + g_ref[...]) + b_ref[...], 0.0)
    o = oin_ref[...] + jnp.dot(h3, wl_ref[2 * d:3 * d, :],
                               preferred_element_type=jnp.float32)
    o = o + bl_ref[...]
    m = jnp.max(o, axis=1, keepdims=True)
    lse = jnp.log(jnp.sum(jnp.exp(o - m), axis=1, keepdims=True)) + m
    o_ref[...] = o - lse


def kernel(x, edge_index, W0, b0, W1, b1, W2, b2, Wl, bl):
    n, d_in = x.shape
    e = edge_index.shape[1]
    d = W0.shape[1]
    d_out = Wl.shape[1]
    n_pad = _cdiv(n, 128 * NS) * (128 * NS)
    bn = 2000
    grid = n // bn

    src = edge_index[0]
    dst = edge_index[1]
    src2 = src.reshape(e // CHUNK, 1, CHUNK)
    dst2 = dst.reshape(e // CHUNK, 1, CHUNK)
    zeros_row = jnp.zeros((n_pad // NS,), jnp.float32)
    zeros_blk = jnp.zeros((n_pad // NS, d), jnp.float32)

    deg2 = _make_deg_kernel(n_pad, e)(dst2, dst, zeros_row)
    degc = (deg2[0, 0, :n] + deg2[1, 0, :n]).reshape(n, 1)

    agg_fn = _make_agg_kernel(n, n_pad, d, e)

    row_blk = lambda i: (i, 0)
    full_w = lambda a, b_: pl.BlockSpec((a, b_), lambda i: (0, 0))
    vec = lambda L: pl.BlockSpec((L,), lambda i: (0,))
    nd_blk = pl.BlockSpec((bn, d), row_blk)
    col_blk = pl.BlockSpec((bn, 1), row_blk)
    agg_blk = pl.BlockSpec((NC, bn, d), lambda i: (0, i, 0))

    o_blk = pl.BlockSpec((bn, d_out), row_blk)
    o_shape = jax.ShapeDtypeStruct((n, d_out), jnp.float32)
    nd_shape = jax.ShapeDtypeStruct((n, d), jnp.float32)

    g0 = pl.pallas_call(
        _pre_body,
        grid=(grid,),
        in_specs=[nd_blk, full_w(d_in, d), col_blk],
        out_specs=nd_blk,
        out_shape=nd_shape,
    )(x, W0, degc)

    agg1 = agg_fn(g0, src2, src, dst, zeros_blk)
    g1, o1 = pl.pallas_call(
        functools.partial(_mid1_body, d),
        grid=(grid,),
        in_specs=[agg_blk, nd_blk, col_blk, vec(d), full_w(d, d),
                  full_w(3 * d, d_out)],
        out_specs=[nd_blk, o_blk],
        out_shape=[nd_shape, o_shape],
    )(agg1, g0, degc, b0, W1, Wl)

    agg2 = agg_fn(g1, src2, src, dst, zeros_blk)
    g2, o2 = pl.pallas_call(
        functools.partial(_mid2_body, d),
        grid=(grid,),
        in_specs=[agg_blk, nd_blk, col_blk, vec(d), full_w(d, d),
                  full_w(3 * d, d_out), o_blk],
        out_specs=[nd_blk, o_blk],
        out_shape=[nd_shape, o_shape],
    )(agg2, g1, degc, b1, W2, Wl, o1)

    agg3 = agg_fn(g2, src2, src, dst, zeros_blk)
    out = pl.pallas_call(
        functools.partial(_post_body, d),
        grid=(grid,),
        in_specs=[agg_blk, nd_blk, col_blk, vec(d),
                  full_w(3 * d, d_out), vec(d_out), o_blk],
        out_specs=o_blk,
        out_shape=o_shape,
    )(agg3, g2, degc, b2, Wl, bl, o2)

    return out


# speedup vs baseline: 29.0763x; 1.0085x over previous
import functools

import jax
import jax.numpy as jnp
from jax import lax
from jax.experimental import pallas as pl
from jax.experimental.pallas import tpu as pltpu
from jax.experimental.pallas import tpu_sc as plsc

NC = 2
NS = 16
LANES = 16
CHUNK = 128


def _cdiv(a, b):
    return (a + b - 1) // b


def _deg_body(n_pad, e, dst2_hbm, dst1_hbm, zeros_hbm, deg_hbm,
              idx_v, ones_v, deg_sh, sem):
    c = lax.axis_index("c")
    s = lax.axis_index("s")
    rows = n_pad // NS

    total_chunks = e // CHUNK
    nw = NC * NS
    w = s * NC + c
    nbase = total_chunks // nw
    rem = total_chunks - nbase * nw

    for k in range(CHUNK // LANES):
        ones_v[pl.ds(k * LANES, LANES)] = jnp.ones((LANES,), jnp.float32)
    pltpu.sync_copy(zeros_hbm, deg_sh.at[pl.ds(s * rows, rows)])
    pltpu.sync_copy(dst2_hbm.at[pl.ds(w * nbase, nbase)],
                    idx_v.at[pl.ds(0, nbase)])

    @pl.when(w < rem)
    def _():
        pltpu.sync_copy(dst1_hbm.at[pl.ds((nbase * nw + w) * CHUNK, CHUNK)],
                        idx_v.at[nbase, 0])

    plsc.subcore_barrier()

    ntot = nbase + 1
    def fire(j, carry):
        @pl.when((j < nbase) | (w < rem))
        def _():
            pltpu.async_copy(ones_v, deg_sh.at[idx_v.at[j, 0]], sem,
                             add=True)
        return carry

    lax.fori_loop(0, ntot, fire, 0)

    def drain(j, carry):
        @pl.when((j < nbase) | (w < rem))
        def _():
            pltpu.make_async_copy(ones_v, deg_sh.at[idx_v.at[j, 0]],
                                  sem).wait()
        return carry

    lax.fori_loop(0, ntot, drain, 0)
    plsc.subcore_barrier()
    pltpu.sync_copy(deg_sh.at[pl.ds(s * rows, rows)],
                    deg_hbm.at[c, 0, pl.ds(s * rows, rows)])


def _make_deg_kernel(n_pad, e):
    mesh = plsc.VectorSubcoreMesh(core_axis_name="c", subcore_axis_name="s")
    nchunk_t = _cdiv(e // CHUNK, NC * NS)
    return pl.kernel(
        functools.partial(_deg_body, n_pad, e),
        out_type=jax.ShapeDtypeStruct((NC, 1, n_pad), jnp.float32),
        mesh=mesh,
        scratch_types=[
            pltpu.VMEM((nchunk_t, 1, CHUNK), jnp.int32),
            pltpu.VMEM((CHUNK,), jnp.float32),
            pltpu.VMEM_SHARED((n_pad,), jnp.float32),
            pltpu.SemaphoreType.DMA,
        ],
    )


def _agg_body(n, n_pad, d, e, g_hbm, src2_hbm, src1_hbm, dst1_hbm, zeros_hbm,
              out_hbm, src_v, dst_a, dst_b, rows0, rows1, acc_sh,
              gsem0, gsem1, ssem0, ssem1):
    c = lax.axis_index("c")
    s = lax.axis_index("s")
    rows = n_pad // NS

    total_chunks = e // CHUNK
    nw = NC * NS
    w = s * NC + c
    nbase = total_chunks // nw
    rem = total_chunks - nbase * nw
    full_tiles = n // rows
    tail = n - full_tiles * rows

    pltpu.async_copy(src2_hbm.at[pl.ds(w * nbase, nbase)],
                     src_v.at[pl.ds(0, nbase)], gsem1)

    @pl.when(w < rem)
    def _():
        pltpu.async_copy(src1_hbm.at[pl.ds((nbase * nw + w) * CHUNK, CHUNK)],
                         src_v.at[nbase, 0], gsem1)

    @pl.when((c == 0) & (s < full_tiles))
    def _():
        pltpu.sync_copy(g_hbm.at[pl.ds(s * rows, rows)],
                        acc_sh.at[pl.ds(s * rows, rows)])

    if tail > 0:
        @pl.when((c == 0) & (s == full_tiles))
        def _():
            pltpu.sync_copy(g_hbm.at[pl.ds(full_tiles * rows, tail)],
                            acc_sh.at[pl.ds(full_tiles * rows, tail)])
            pltpu.sync_copy(
                zeros_hbm.at[pl.ds(0, rows - tail)],
                acc_sh.at[pl.ds(full_tiles * rows + tail, rows - tail)])

    zero_cond = ((c != 0) | (s > full_tiles)) if tail > 0 else (
        (c != 0) | (s >= full_tiles))

    @pl.when(zero_cond)
    def _():
        pltpu.sync_copy(zeros_hbm, acc_sh.at[pl.ds(s * rows, rows)])

    pltpu.make_async_copy(src2_hbm.at[pl.ds(w * nbase, nbase)],
                          src_v.at[pl.ds(0, nbase)], gsem1).wait()

    @pl.when(w < rem)
    def _():
        pltpu.make_async_copy(
            src1_hbm.at[pl.ds((nbase * nw + w) * CHUNK, CHUNK)],
            src_v.at[nbase, 0], gsem1).wait()

    plsc.subcore_barrier()

    rbufs = (rows0, rows1)
    dbufs = (dst_a, dst_b)
    gsems = (gsem0, gsem1)
    ssems = (ssem0, ssem1)

    def chunk_idx(j):
        return jnp.where(j < nbase, w * nbase + j, nbase * nw + w)

    def fire(j, b):
        pltpu.async_copy(g_hbm.at[src_v.at[j, 0]], rbufs[b], gsems[b])
        pltpu.async_copy(dst1_hbm.at[pl.ds(chunk_idx(j) * CHUNK, CHUNK)],
                         dbufs[b], gsems[b])

    def wait_gather(j, b):
        pltpu.make_async_copy(g_hbm.at[src_v.at[j, 0]], rbufs[b],
                              gsems[b]).wait()
        pltpu.make_async_copy(dst1_hbm.at[pl.ds(0, CHUNK)], dbufs[b],
                              gsems[b]).wait()

    def wait_scatter(b):
        pltpu.make_async_copy(rbufs[b], acc_sh.at[dbufs[b]], ssems[b]).wait()

    fire(0, 0)

    def pair(jj, carry):
        for b in (0, 1):
            j = 2 * jj + b
            nxt = j + 1

            if b == 0:
                @pl.when(jj > 0)
                def _():
                    wait_scatter(1)
            else:
                wait_scatter(0)

            @pl.when((nxt < nbase) | ((nxt == nbase) & (w < rem)))
            def _():
                fire(nxt, 1 - b)

            wait_gather(j, b)
            pltpu.async_copy(rbufs[b], acc_sh.at[dbufs[b]], ssems[b],
                             add=True)
        return carry

    assert nbase % 2 == 0
    lax.fori_loop(0, nbase // 2, pair, 0)

    @pl.when(w < rem)
    def _():
        wait_gather(nbase, 0)
        pltpu.async_copy(rbufs[0], acc_sh.at[dbufs[0]], ssems[0], add=True)
        wait_scatter(0)

    wait_scatter(1)
    plsc.subcore_barrier()
    pltpu.sync_copy(acc_sh.at[pl.ds(s * rows, rows)],
                    out_hbm.at[c, pl.ds(s * rows, rows)])


def _make_agg_kernel(n, n_pad, d, e):
    mesh = plsc.VectorSubcoreMesh(core_axis_name="c", subcore_axis_name="s")
    nchunk_t = _cdiv(e // CHUNK, NC * NS)
    return pl.kernel(
        functools.partial(_agg_body, n, n_pad, d, e),
        out_type=jax.ShapeDtypeStruct((NC, n_pad, d), jnp.float32),
        mesh=mesh,
        scratch_types=[
            pltpu.VMEM((nchunk_t, 1, CHUNK), jnp.int32),
            pltpu.VMEM((CHUNK,), jnp.int32),
            pltpu.VMEM((CHUNK,), jnp.int32),
            pltpu.VMEM((CHUNK, d), jnp.float32),
            pltpu.VMEM((CHUNK, d), jnp.float32),
            pltpu.VMEM_SHARED((n_pad, d), jnp.float32),
            pltpu.SemaphoreType.DMA,
            pltpu.SemaphoreType.DMA,
            pltpu.SemaphoreType.DMA,
            pltpu.SemaphoreType.DMA,
        ],
    )


def _pre_body(x_ref, w_ref, degc_ref, g_ref):
    dis = lax.rsqrt(1.0 + degc_ref[...])
    t = jnp.dot(x_ref[...], w_ref[...], preferred_element_type=jnp.float32)
    g_ref[...] = dis * t


def _mid1_body(d, agg_ref, degc_ref, b_ref, w_ref, wl_ref,
               gn_ref, o_ref):
    dis = lax.rsqrt(1.0 + degc_ref[...])
    a = agg_ref[0] + agg_ref[1]
    h = jnp.maximum(dis * a + b_ref[...], 0.0)
    tn = jnp.dot(h, w_ref[...], preferred_element_type=jnp.float32)
    gn_ref[...] = dis * tn
    o_ref[...] = jnp.dot(h, wl_ref[0:d, :],
                         preferred_element_type=jnp.float32)


def _mid2_body(d, agg_ref, degc_ref, b_ref, w_ref, wl_ref, oin_ref,
               gn_ref, o_ref):
    dis = lax.rsqrt(1.0 + degc_ref[...])
    a = agg_ref[0] + agg_ref[1]
    h = jnp.maximum(dis * a + b_ref[...], 0.0)
    tn = jnp.dot(h, w_ref[...], preferred_element_type=jnp.float32)
    gn_ref[...] = dis * tn
    o_ref[...] = oin_ref[...] + jnp.dot(h, wl_ref[d:2 * d, :],
                                        preferred_element_type=jnp.float32)


def _post_body(d, agg_ref, degc_ref, b_ref, wl_ref, bl_ref, oin_ref,
               o_ref):
    dis = lax.rsqrt(1.0 + degc_ref[...])
    a = agg_ref[0] + agg_ref[1]
    h3 = jnp.maximum(dis * a + b_ref[...], 0.0)
    o = oin_ref[...] + jnp.dot(h3, wl_ref[2 * d:3 * d, :],
                               preferred_element_type=jnp.float32)
    o = o + bl_ref[...]
    m = jnp.max(o, axis=1, keepdims=True)
    lse = jnp.log(jnp.sum(jnp.exp(o - m), axis=1, keepdims=True)) + m
    o_ref[...] = o - lse


def kernel(x, edge_index, W0, b0, W1, b1, W2, b2, Wl, bl):
    n, d_in = x.shape
    e = edge_index.shape[1]
    d = W0.shape[1]
    d_out = Wl.shape[1]
    n_pad = _cdiv(n, 128 * NS) * (128 * NS)
    bn = 2000
    grid = n // bn

    src = edge_index[0]
    dst = edge_index[1]
    src2 = src.reshape(e // CHUNK, 1, CHUNK)
    dst2 = dst.reshape(e // CHUNK, 1, CHUNK)
    zeros_row = jnp.zeros((n_pad // NS,), jnp.float32)
    zeros_blk = jnp.zeros((n_pad // NS, d), jnp.float32)

    deg2 = _make_deg_kernel(n_pad, e)(dst2, dst, zeros_row)
    degc = (deg2[0, 0, :n] + deg2[1, 0, :n]).reshape(n, 1)

    agg_fn = _make_agg_kernel(n, n_pad, d, e)

    row_blk = lambda i: (i, 0)
    full_w = lambda a, b_: pl.BlockSpec((a, b_), lambda i: (0, 0))
    vec = lambda L: pl.BlockSpec((L,), lambda i: (0,))
    nd_blk = pl.BlockSpec((bn, d), row_blk)
    col_blk = pl.BlockSpec((bn, 1), row_blk)
    agg_blk = pl.BlockSpec((NC, bn, d), lambda i: (0, i, 0))

    o_blk = pl.BlockSpec((bn, d_out), row_blk)
    o_shape = jax.ShapeDtypeStruct((n, d_out), jnp.float32)
    nd_shape = jax.ShapeDtypeStruct((n, d), jnp.float32)

    g0 = pl.pallas_call(
        _pre_body,
        grid=(grid,),
        in_specs=[nd_blk, full_w(d_in, d), col_blk],
        out_specs=nd_blk,
        out_shape=nd_shape,
    )(x, W0, degc)

    agg1 = agg_fn(g0, src2, src, dst, zeros_blk)
    g1, o1 = pl.pallas_call(
        functools.partial(_mid1_body, d),
        grid=(grid,),
        in_specs=[agg_blk, col_blk, vec(d), full_w(d, d),
                  full_w(3 * d, d_out)],
        out_specs=[nd_blk, o_blk],
        out_shape=[nd_shape, o_shape],
    )(agg1, degc, b0, W1, Wl)

    agg2 = agg_fn(g1, src2, src, dst, zeros_blk)
    g2, o2 = pl.pallas_call(
        functools.partial(_mid2_body, d),
        grid=(grid,),
        in_specs=[agg_blk, col_blk, vec(d), full_w(d, d),
                  full_w(3 * d, d_out), o_blk],
        out_specs=[nd_blk, o_blk],
        out_shape=[nd_shape, o_shape],
    )(agg2, degc, b1, W2, Wl, o1)

    agg3 = agg_fn(g2, src2, src, dst, zeros_blk)
    out = pl.pallas_call(
        functools.partial(_post_body, d),
        grid=(grid,),
        in_specs=[agg_blk, col_blk, vec(d),
                  full_w(3 * d, d_out), vec(d_out), o_blk],
        out_specs=o_blk,
        out_shape=o_shape,
    )(agg3, degc, b2, Wl, bl, o2)

    return out
